# Initial kernel scaffold; baseline (speedup 1.0000x reference)
import jax
import jax.numpy as jnp
from jax.experimental import pallas as pl


def kernel(x, edge_index, W1, b1, W2, b2, W3, b3):
    raise NotImplementedError("write your pallas kernel here")


# speedup vs baseline: 5.1829x; 5.1829x over previous
import functools

import jax
import jax.numpy as jnp
from jax import lax
from jax.experimental import pallas as pl
from jax.experimental.pallas import tpu as pltpu
from jax.experimental.pallas import tpu_sc as plsc

N = 10000
E = 320000
D_IN = 128
D_H = 128
D_OUT = 40

NC = 2
NS = 16
NW = NC * NS
L = 16

NPAD = 10240
E_PER_W = E // NW


def _sc_degrees(src, dst):
    CH = 2000
    NT = NPAD // NS

    @functools.partial(
        pl.kernel,
        out_type=jax.ShapeDtypeStruct((NC, 2, NPAD), jnp.float32),
        mesh=plsc.VectorSubcoreMesh(core_axis_name="c", subcore_axis_name="s"),
        scratch_types=[
            pltpu.VMEM((CH,), jnp.int32),
            pltpu.VMEM((CH,), jnp.float32),
            pltpu.VMEM((NT,), jnp.float32),
            pltpu.VMEM_SHARED((NPAD,), jnp.float32),
            pltpu.VMEM_SHARED((NPAD,), jnp.float32),
        ],
    )
    def k(src_hbm, dst_hbm, out_hbm, idxv, onesv, tmpv, acc_s, acc_d):
        cid = lax.axis_index("c")
        sid = lax.axis_index("s")
        wid = cid * NS + sid

        def fill(i, _):
            onesv[pl.ds(i * L, L)] = jnp.full((L,), 1.0, jnp.float32)
            tmpv[pl.ds((i % (NT // L)) * L, L)] = jnp.zeros((L,), jnp.float32)
            return 0

        lax.fori_loop(0, CH // L, fill, 0)

        pltpu.sync_copy(tmpv, acc_s.at[pl.ds(sid * NT, NT)])
        pltpu.sync_copy(tmpv, acc_d.at[pl.ds(sid * NT, NT)])
        plsc.subcore_barrier()

        ebase = wid * E_PER_W

        def chunk(c, _):
            base = ebase + c * CH
            pltpu.sync_copy(src_hbm.at[pl.ds(base, CH)], idxv)
            pltpu.sync_copy(onesv, acc_s.at[idxv], add=True)
            pltpu.sync_copy(dst_hbm.at[pl.ds(base, CH)], idxv)
            pltpu.sync_copy(onesv, acc_d.at[idxv], add=True)
            return 0

        lax.fori_loop(0, E_PER_W // CH, chunk, 0)
        plsc.subcore_barrier()

        pltpu.sync_copy(acc_s.at[pl.ds(sid * NT, NT)], tmpv)
        pltpu.sync_copy(tmpv, out_hbm.at[cid, 0, pl.ds(sid * NT, NT)])
        pltpu.sync_copy(acc_d.at[pl.ds(sid * NT, NT)], tmpv)
        pltpu.sync_copy(tmpv, out_hbm.at[cid, 1, pl.ds(sid * NT, NT)])

    return k(src, dst)


def _sc_spmm(h, src, dst, d):
    CH = 80
    RT = NPAD // NS
    ZR = 128

    @functools.partial(
        pl.kernel,
        out_type=jax.ShapeDtypeStruct((NC, NPAD, d), jnp.float32),
        mesh=plsc.VectorSubcoreMesh(core_axis_name="c", subcore_axis_name="s"),
        scratch_types=[
            pltpu.VMEM((CH,), jnp.int32),
            pltpu.VMEM((CH,), jnp.int32),
            pltpu.VMEM((CH, d), jnp.float32),
            pltpu.VMEM((ZR, d), jnp.float32),
            pltpu.VMEM_SHARED((NPAD, d), jnp.float32),
            pltpu.SemaphoreType.DMA,
        ],
    )
    def k(h_hbm, src_hbm, dst_hbm, out_hbm, sidx, didx, rows, zbuf, acc, sem):
        cid = lax.axis_index("c")
        sid = lax.axis_index("s")
        wid = cid * NS + sid

        def zfill(i, _):
            r = i // (d // L)
            c = i % (d // L)
            zbuf[r, pl.ds(c * L, L)] = jnp.zeros((L,), jnp.float32)
            return 0

        lax.fori_loop(0, (ZR * d) // L, zfill, 0)

        row0 = sid * RT
        for kk in range(RT // ZR):
            pltpu.sync_copy(zbuf, acc.at[pl.ds(row0 + kk * ZR, ZR)])
        plsc.subcore_barrier()

        ebase = wid * E_PER_W

        def chunk(c, _):
            base = ebase + c * CH
            pltpu.sync_copy(src_hbm.at[pl.ds(base, CH)], sidx)
            pltpu.sync_copy(dst_hbm.at[pl.ds(base, CH)], didx)
            pltpu.async_copy(h_hbm.at[sidx], rows, sem).wait()
            pltpu.sync_copy(rows, acc.at[didx], add=True)
            return 0

        lax.fori_loop(0, E_PER_W // CH, chunk, 0)
        plsc.subcore_barrier()

        for kk in range(RT // ZR):
            pltpu.sync_copy(acc.at[pl.ds(row0 + kk * ZR, ZR)], zbuf)
            pltpu.sync_copy(zbuf, out_hbm.at[cid, pl.ds(row0 + kk * ZR, ZR)])

    return k(h, src, dst)


def _tc_norms(deg_partials):
    def k(dp_ref, o_ref):
        deg = dp_ref[0] + dp_ref[1]
        o_ref[...] = lax.rsqrt(jnp.maximum(deg, 1.0))

    return pl.pallas_call(
        k,
        out_shape=jax.ShapeDtypeStruct((2, NPAD), jnp.float32),
    )(deg_partials)


def _tc_pre_matmul(x, ns, w):
    R = 1000

    def k(x_ref, ns_ref, w_ref, o_ref):
        o_ref[...] = jnp.dot(x_ref[...] * ns_ref[...], w_ref[...],
                             preferred_element_type=jnp.float32)

    d_in, d_out = w.shape
    return pl.pallas_call(
        k,
        grid=(N // R,),
        in_specs=[
            pl.BlockSpec((R, d_in), lambda i: (i, 0)),
            pl.BlockSpec((R, 1), lambda i: (i, 0)),
            pl.BlockSpec((d_in, d_out), lambda i: (0, 0)),
        ],
        out_specs=pl.BlockSpec((R, d_out), lambda i: (i, 0)),
        out_shape=jax.ShapeDtypeStruct((N, d_out), jnp.float32),
    )(x, ns, w)


def _tc_mid(partials, nd, ns, b, w):
    R = 1000

    def k(p_ref, nd_ref, ns_ref, b_ref, w_ref, o_ref):
        t = (p_ref[0] + p_ref[1]) * nd_ref[...] + b_ref[...]
        t = jnp.maximum(t, 0.0) * ns_ref[...]
        o_ref[...] = jnp.dot(t, w_ref[...], preferred_element_type=jnp.float32)

    d_in, d_out = w.shape
    return pl.pallas_call(
        k,
        grid=(N // R,),
        in_specs=[
            pl.BlockSpec((NC, R, d_in), lambda i: (0, i, 0)),
            pl.BlockSpec((R, 1), lambda i: (i, 0)),
            pl.BlockSpec((R, 1), lambda i: (i, 0)),
            pl.BlockSpec((1, d_in), lambda i: (0, 0)),
            pl.BlockSpec((d_in, d_out), lambda i: (0, 0)),
        ],
        out_specs=pl.BlockSpec((R, d_out), lambda i: (i, 0)),
        out_shape=jax.ShapeDtypeStruct((N, d_out), jnp.float32),
    )(partials, nd, ns, b, w)


def _tc_elem(partials, nd, ns, b):
    R = 1000

    def k(p_ref, nd_ref, ns_ref, b_ref, o_ref):
        t = (p_ref[0] + p_ref[1]) * nd_ref[...] + b_ref[...]
        o_ref[...] = jnp.maximum(t, 0.0) * ns_ref[...]

    return pl.pallas_call(
        k,
        grid=(N // R,),
        in_specs=[
            pl.BlockSpec((NC, R, D_H), lambda i: (0, i, 0)),
            pl.BlockSpec((R, 1), lambda i: (i, 0)),
            pl.BlockSpec((R, 1), lambda i: (i, 0)),
            pl.BlockSpec((1, D_H), lambda i: (0, 0)),
        ],
        out_specs=pl.BlockSpec((R, D_H), lambda i: (i, 0)),
        out_shape=jax.ShapeDtypeStruct((N, D_H), jnp.float32),
    )(partials, nd, ns, b)


def _tc_final_matmul(partials, nd, b, w):
    R = 1000

    def k(p_ref, nd_ref, b_ref, w_ref, o_ref):
        t = (p_ref[0] + p_ref[1]) * nd_ref[...]
        o_ref[...] = jnp.dot(t, w_ref[...],
                             preferred_element_type=jnp.float32) + b_ref[...]

    d_in, d_out = w.shape
    return pl.pallas_call(
        k,
        grid=(N // R,),
        in_specs=[
            pl.BlockSpec((NC, R, d_in), lambda i: (0, i, 0)),
            pl.BlockSpec((R, 1), lambda i: (i, 0)),
            pl.BlockSpec((1, d_out), lambda i: (0, 0)),
            pl.BlockSpec((d_in, d_out), lambda i: (0, 0)),
        ],
        out_specs=pl.BlockSpec((R, d_out), lambda i: (i, 0)),
        out_shape=jax.ShapeDtypeStruct((N, d_out), jnp.float32),
    )(partials, nd, b, w)


def kernel(x, edge_index, W1, b1, W2, b2, W3, b3):
    src = edge_index[0]
    dst = edge_index[1]

    deg_partials = _sc_degrees(src, dst)
    norms = _tc_norms(deg_partials)
    ns = norms[0, :N, None]
    nd = norms[1, :N, None]

    h = _tc_pre_matmul(x, ns, W1)
    p = _sc_spmm(h, src, dst, D_H)[:, :N]
    h = _tc_mid(p, nd, ns, b1[None, :], W2)
    p = _sc_spmm(h, src, dst, D_H)[:, :N]
    h = _tc_elem(p, nd, ns, b2[None, :])
    p = _sc_spmm(h, src, dst, D_H)[:, :N]
    return _tc_final_matmul(p, nd, b3[None, :], W3)


# speedup vs baseline: 9.1533x; 1.7661x over previous
import functools

import jax
import jax.numpy as jnp
from jax import lax
from jax.experimental import pallas as pl
from jax.experimental.pallas import tpu as pltpu
from jax.experimental.pallas import tpu_sc as plsc

N = 10000
E = 320000
D_IN = 128
D_H = 128
D_OUT = 40

NC = 2
NS = 16
NW = NC * NS
L = 16

NPAD = 10240
E_PER_W = E // NW


def _sc_degrees(src, dst):
    CH = 2000
    NT = NPAD // NS

    @functools.partial(
        pl.kernel,
        out_type=jax.ShapeDtypeStruct((NC, 2, NPAD), jnp.float32),
        mesh=plsc.VectorSubcoreMesh(core_axis_name="c", subcore_axis_name="s"),
        scratch_types=[
            pltpu.VMEM((CH,), jnp.int32),
            pltpu.VMEM((CH,), jnp.float32),
            pltpu.VMEM((NT,), jnp.float32),
            pltpu.VMEM_SHARED((NPAD,), jnp.float32),
            pltpu.VMEM_SHARED((NPAD,), jnp.float32),
        ],
    )
    def k(src_hbm, dst_hbm, out_hbm, idxv, onesv, tmpv, acc_s, acc_d):
        cid = lax.axis_index("c")
        sid = lax.axis_index("s")
        wid = cid * NS + sid

        def fill(i, _):
            onesv[pl.ds(i * L, L)] = jnp.full((L,), 1.0, jnp.float32)
            tmpv[pl.ds((i % (NT // L)) * L, L)] = jnp.zeros((L,), jnp.float32)
            return 0

        lax.fori_loop(0, CH // L, fill, 0)

        pltpu.sync_copy(tmpv, acc_s.at[pl.ds(sid * NT, NT)])
        pltpu.sync_copy(tmpv, acc_d.at[pl.ds(sid * NT, NT)])
        plsc.subcore_barrier()

        ebase = wid * E_PER_W

        def chunk(c, _):
            base = ebase + c * CH
            pltpu.sync_copy(src_hbm.at[pl.ds(base, CH)], idxv)
            pltpu.sync_copy(onesv, acc_s.at[idxv], add=True)
            pltpu.sync_copy(dst_hbm.at[pl.ds(base, CH)], idxv)
            pltpu.sync_copy(onesv, acc_d.at[idxv], add=True)
            return 0

        lax.fori_loop(0, E_PER_W // CH, chunk, 0)
        plsc.subcore_barrier()

        pltpu.sync_copy(acc_s.at[pl.ds(sid * NT, NT)], tmpv)
        pltpu.sync_copy(tmpv, out_hbm.at[cid, 0, pl.ds(sid * NT, NT)])
        pltpu.sync_copy(acc_d.at[pl.ds(sid * NT, NT)], tmpv)
        pltpu.sync_copy(tmpv, out_hbm.at[cid, 1, pl.ds(sid * NT, NT)])

    return k(src, dst)


def _sc_spmm(h, src, dst, d):
    CH = 80
    RT = NPAD // NS
    ZR = 128

    NITER = E_PER_W // CH

    @functools.partial(
        pl.kernel,
        out_type=jax.ShapeDtypeStruct((NC, NPAD, d), jnp.float32),
        mesh=plsc.VectorSubcoreMesh(core_axis_name="c", subcore_axis_name="s"),
        scratch_types=[
            pltpu.VMEM((CH,), jnp.int32),
            pltpu.VMEM((CH,), jnp.int32),
            pltpu.VMEM((CH,), jnp.int32),
            pltpu.VMEM((CH,), jnp.int32),
            pltpu.VMEM((CH, d), jnp.float32),
            pltpu.VMEM((CH, d), jnp.float32),
            pltpu.VMEM((ZR, d), jnp.float32),
            pltpu.VMEM_SHARED((NPAD, d), jnp.float32),
            pltpu.SemaphoreType.DMA,
            pltpu.SemaphoreType.DMA,
            pltpu.SemaphoreType.DMA,
        ],
    )
    def k(h_hbm, src_hbm, dst_hbm, out_hbm,
          sidx0, didx0, sidx1, didx1, rows0, rows1, zbuf, acc,
          semi, semg0, semg1):
        cid = lax.axis_index("c")
        sid = lax.axis_index("s")
        wid = cid * NS + sid

        def zfill(i, _):
            r = i // (d // L)
            c = i % (d // L)
            zbuf[r, pl.ds(c * L, L)] = jnp.zeros((L,), jnp.float32)
            return 0

        lax.fori_loop(0, (ZR * d) // L, zfill, 0)

        row0 = sid * RT
        for kk in range(RT // ZR):
            pltpu.sync_copy(zbuf, acc.at[pl.ds(row0 + kk * ZR, ZR)])
        plsc.subcore_barrier()

        ebase = wid * E_PER_W

        pltpu.async_copy(src_hbm.at[pl.ds(ebase, CH)], sidx0, semi)
        pltpu.async_copy(dst_hbm.at[pl.ds(ebase, CH)], didx0, semi)
        pltpu.make_async_copy(src_hbm.at[pl.ds(ebase, CH)], sidx0, semi).wait()
        pltpu.make_async_copy(dst_hbm.at[pl.ds(ebase, CH)], didx0, semi).wait()
        pltpu.async_copy(h_hbm.at[sidx0], rows0, semg0)

        def step(i, sidx, didx, rows, semg, sidxn, didxn, rowsn, semgn):
            not_last = i < NITER - 1

            @pl.when(not_last)
            def _():
                base = ebase + (i + 1) * CH
                pltpu.async_copy(src_hbm.at[pl.ds(base, CH)], sidxn, semi)
                pltpu.async_copy(dst_hbm.at[pl.ds(base, CH)], didxn, semi)

            pltpu.make_async_copy(h_hbm.at[pl.ds(0, CH)], rows, semg).wait()

            @pl.when(not_last)
            def _():
                pltpu.make_async_copy(src_hbm.at[pl.ds(0, CH)], sidxn, semi).wait()
                pltpu.make_async_copy(dst_hbm.at[pl.ds(0, CH)], didxn, semi).wait()
                pltpu.async_copy(h_hbm.at[sidxn], rowsn, semgn)

            pltpu.sync_copy(rows, acc.at[didx], add=True)

        def body(i, _):
            @pl.when(i % 2 == 0)
            def _():
                step(i, sidx0, didx0, rows0, semg0, sidx1, didx1, rows1, semg1)

            @pl.when(i % 2 == 1)
            def _():
                step(i, sidx1, didx1, rows1, semg1, sidx0, didx0, rows0, semg0)

            return 0

        lax.fori_loop(0, NITER, body, 0)
        plsc.subcore_barrier()

        for kk in range(RT // ZR):
            pltpu.sync_copy(acc.at[pl.ds(row0 + kk * ZR, ZR)], zbuf)
            pltpu.sync_copy(zbuf, out_hbm.at[cid, pl.ds(row0 + kk * ZR, ZR)])

    return k(h, src, dst)


def _tc_norms(deg_partials):
    def k(dp_ref, o_ref):
        deg = dp_ref[0] + dp_ref[1]
        o_ref[...] = lax.rsqrt(jnp.maximum(deg, 1.0))

    return pl.pallas_call(
        k,
        out_shape=jax.ShapeDtypeStruct((2, NPAD), jnp.float32),
    )(deg_partials)


def _tc_pre_matmul(x, ns, w):
    R = 1000

    def k(x_ref, ns_ref, w_ref, o_ref):
        o_ref[...] = jnp.dot(x_ref[...] * ns_ref[...], w_ref[...],
                             preferred_element_type=jnp.float32)

    d_in, d_out = w.shape
    return pl.pallas_call(
        k,
        grid=(N // R,),
        in_specs=[
            pl.BlockSpec((R, d_in), lambda i: (i, 0)),
            pl.BlockSpec((R, 1), lambda i: (i, 0)),
            pl.BlockSpec((d_in, d_out), lambda i: (0, 0)),
        ],
        out_specs=pl.BlockSpec((R, d_out), lambda i: (i, 0)),
        out_shape=jax.ShapeDtypeStruct((N, d_out), jnp.float32),
    )(x, ns, w)


def _tc_mid(partials, nd, ns, b, w):
    R = 1000

    def k(p_ref, nd_ref, ns_ref, b_ref, w_ref, o_ref):
        t = (p_ref[0] + p_ref[1]) * nd_ref[...] + b_ref[...]
        t = jnp.maximum(t, 0.0) * ns_ref[...]
        o_ref[...] = jnp.dot(t, w_ref[...], preferred_element_type=jnp.float32)

    d_in, d_out = w.shape
    return pl.pallas_call(
        k,
        grid=(N // R,),
        in_specs=[
            pl.BlockSpec((NC, R, d_in), lambda i: (0, i, 0)),
            pl.BlockSpec((R, 1), lambda i: (i, 0)),
            pl.BlockSpec((R, 1), lambda i: (i, 0)),
            pl.BlockSpec((1, d_in), lambda i: (0, 0)),
            pl.BlockSpec((d_in, d_out), lambda i: (0, 0)),
        ],
        out_specs=pl.BlockSpec((R, d_out), lambda i: (i, 0)),
        out_shape=jax.ShapeDtypeStruct((N, d_out), jnp.float32),
    )(partials, nd, ns, b, w)


def _tc_elem(partials, nd, ns, b):
    R = 1000

    def k(p_ref, nd_ref, ns_ref, b_ref, o_ref):
        t = (p_ref[0] + p_ref[1]) * nd_ref[...] + b_ref[...]
        o_ref[...] = jnp.maximum(t, 0.0) * ns_ref[...]

    return pl.pallas_call(
        k,
        grid=(N // R,),
        in_specs=[
            pl.BlockSpec((NC, R, D_H), lambda i: (0, i, 0)),
            pl.BlockSpec((R, 1), lambda i: (i, 0)),
            pl.BlockSpec((R, 1), lambda i: (i, 0)),
            pl.BlockSpec((1, D_H), lambda i: (0, 0)),
        ],
        out_specs=pl.BlockSpec((R, D_H), lambda i: (i, 0)),
        out_shape=jax.ShapeDtypeStruct((N, D_H), jnp.float32),
    )(partials, nd, ns, b)


def _tc_final_matmul(partials, nd, b, w):
    R = 1000

    def k(p_ref, nd_ref, b_ref, w_ref, o_ref):
        t = (p_ref[0] + p_ref[1]) * nd_ref[...]
        o_ref[...] = jnp.dot(t, w_ref[...],
                             preferred_element_type=jnp.float32) + b_ref[...]

    d_in, d_out = w.shape
    return pl.pallas_call(
        k,
        grid=(N // R,),
        in_specs=[
            pl.BlockSpec((NC, R, d_in), lambda i: (0, i, 0)),
            pl.BlockSpec((R, 1), lambda i: (i, 0)),
            pl.BlockSpec((1, d_out), lambda i: (0, 0)),
            pl.BlockSpec((d_in, d_out), lambda i: (0, 0)),
        ],
        out_specs=pl.BlockSpec((R, d_out), lambda i: (i, 0)),
        out_shape=jax.ShapeDtypeStruct((N, d_out), jnp.float32),
    )(partials, nd, b, w)


def kernel(x, edge_index, W1, b1, W2, b2, W3, b3):
    src = edge_index[0]
    dst = edge_index[1]

    deg_partials = _sc_degrees(src, dst)
    norms = _tc_norms(deg_partials)
    ns = norms[0, :N, None]
    nd = norms[1, :N, None]

    h = _tc_pre_matmul(x, ns, W1)
    p = _sc_spmm(h, src, dst, D_H)[:, :N]
    h = _tc_mid(p, nd, ns, b1[None, :], W2)
    p = _sc_spmm(h, src, dst, D_H)[:, :N]
    h = _tc_elem(p, nd, ns, b2[None, :])
    p = _sc_spmm(h, src, dst, D_H)[:, :N]
    return _tc_final_matmul(p, nd, b3[None, :], W3)


# speedup vs baseline: 9.1765x; 1.0025x over previous
import functools

import jax
import jax.numpy as jnp
from jax import lax
from jax.experimental import pallas as pl
from jax.experimental.pallas import tpu as pltpu
from jax.experimental.pallas import tpu_sc as plsc

N = 10000
E = 320000
D_IN = 128
D_H = 128
D_OUT = 40

NC = 2
NS = 16
NW = NC * NS
L = 16

NPAD = 10240
E_PER_W = E // NW


def _sc_degrees(src, dst):
    CH = 2000
    NT = NPAD // NS

    @functools.partial(
        pl.kernel,
        out_type=jax.ShapeDtypeStruct((NC, 2, NPAD), jnp.float32),
        mesh=plsc.VectorSubcoreMesh(core_axis_name="c", subcore_axis_name="s"),
        scratch_types=[
            pltpu.VMEM((CH,), jnp.int32),
            pltpu.VMEM((CH,), jnp.float32),
            pltpu.VMEM((NT,), jnp.float32),
            pltpu.VMEM_SHARED((NPAD,), jnp.float32),
            pltpu.VMEM_SHARED((NPAD,), jnp.float32),
        ],
    )
    def k(src_hbm, dst_hbm, out_hbm, idxv, onesv, tmpv, acc_s, acc_d):
        cid = lax.axis_index("c")
        sid = lax.axis_index("s")
        wid = cid * NS + sid

        def fill(i, _):
            onesv[pl.ds(i * L, L)] = jnp.full((L,), 1.0, jnp.float32)
            tmpv[pl.ds((i % (NT // L)) * L, L)] = jnp.zeros((L,), jnp.float32)
            return 0

        lax.fori_loop(0, CH // L, fill, 0)

        pltpu.sync_copy(tmpv, acc_s.at[pl.ds(sid * NT, NT)])
        pltpu.sync_copy(tmpv, acc_d.at[pl.ds(sid * NT, NT)])
        plsc.subcore_barrier()

        ebase = wid * E_PER_W

        def chunk(c, _):
            base = ebase + c * CH
            pltpu.sync_copy(src_hbm.at[pl.ds(base, CH)], idxv)
            pltpu.sync_copy(onesv, acc_s.at[idxv], add=True)
            pltpu.sync_copy(dst_hbm.at[pl.ds(base, CH)], idxv)
            pltpu.sync_copy(onesv, acc_d.at[idxv], add=True)
            return 0

        lax.fori_loop(0, E_PER_W // CH, chunk, 0)
        plsc.subcore_barrier()

        pltpu.sync_copy(acc_s.at[pl.ds(sid * NT, NT)], tmpv)
        pltpu.sync_copy(tmpv, out_hbm.at[cid, 0, pl.ds(sid * NT, NT)])
        pltpu.sync_copy(acc_d.at[pl.ds(sid * NT, NT)], tmpv)
        pltpu.sync_copy(tmpv, out_hbm.at[cid, 1, pl.ds(sid * NT, NT)])

    return k(src, dst)


def _sc_spmm(h, src, dst, d):
    CH = 80
    RT = NPAD // NS
    ZR = 128

    NITER = E_PER_W // CH

    @functools.partial(
        pl.kernel,
        out_type=jax.ShapeDtypeStruct((NC, NPAD, d), jnp.float32),
        mesh=plsc.VectorSubcoreMesh(core_axis_name="c", subcore_axis_name="s"),
        scratch_types=[
            pltpu.VMEM((CH,), jnp.int32),
            pltpu.VMEM((CH,), jnp.int32),
            pltpu.VMEM((CH,), jnp.int32),
            pltpu.VMEM((CH,), jnp.int32),
            pltpu.VMEM((CH, d), jnp.float32),
            pltpu.VMEM((CH, d), jnp.float32),
            pltpu.VMEM((ZR, d), jnp.float32),
            pltpu.VMEM_SHARED((NPAD, d), jnp.float32),
            pltpu.SemaphoreType.DMA,
            pltpu.SemaphoreType.DMA,
            pltpu.SemaphoreType.DMA,
            pltpu.SemaphoreType.DMA,
            pltpu.SemaphoreType.DMA,
        ],
    )
    def k(h_hbm, src_hbm, dst_hbm, out_hbm,
          sidx0, didx0, sidx1, didx1, rows0, rows1, zbuf, acc,
          semi, semg0, semg1, sems0, sems1):
        cid = lax.axis_index("c")
        sid = lax.axis_index("s")
        wid = cid * NS + sid

        def zfill(i, _):
            r = i // (d // L)
            c = i % (d // L)
            zbuf[r, pl.ds(c * L, L)] = jnp.zeros((L,), jnp.float32)
            return 0

        lax.fori_loop(0, (ZR * d) // L, zfill, 0)

        row0 = sid * RT
        for kk in range(RT // ZR):
            pltpu.sync_copy(zbuf, acc.at[pl.ds(row0 + kk * ZR, ZR)])
        plsc.subcore_barrier()

        ebase = wid * E_PER_W

        pltpu.async_copy(src_hbm.at[pl.ds(ebase, CH)], sidx0, semi)
        pltpu.async_copy(dst_hbm.at[pl.ds(ebase, CH)], didx0, semi)
        pltpu.make_async_copy(src_hbm.at[pl.ds(ebase, CH)], sidx0, semi).wait()
        pltpu.make_async_copy(dst_hbm.at[pl.ds(ebase, CH)], didx0, semi).wait()
        pltpu.async_copy(h_hbm.at[sidx0], rows0, semg0)

        def step(i, sidx, didx, rows, semg, sems,
                 sidxn, didxn, rowsn, semgn, semsn):
            not_last = i < NITER - 1

            @pl.when(not_last)
            def _():
                base = ebase + (i + 1) * CH
                pltpu.async_copy(src_hbm.at[pl.ds(base, CH)], sidxn, semi)
                pltpu.async_copy(dst_hbm.at[pl.ds(base, CH)], didxn, semi)

            pltpu.make_async_copy(h_hbm.at[pl.ds(0, CH)], rows, semg).wait()

            @pl.when(not_last)
            def _():
                @pl.when(i >= 1)
                def _():
                    pltpu.make_async_copy(
                        rowsn, acc.at[pl.ds(0, CH)], semsn).wait()

                pltpu.make_async_copy(src_hbm.at[pl.ds(0, CH)], sidxn, semi).wait()
                pltpu.make_async_copy(dst_hbm.at[pl.ds(0, CH)], didxn, semi).wait()
                pltpu.async_copy(h_hbm.at[sidxn], rowsn, semgn)

            pltpu.async_copy(rows, acc.at[didx], sems, add=True)

        def body(i, _):
            @pl.when(i % 2 == 0)
            def _():
                step(i, sidx0, didx0, rows0, semg0, sems0,
                     sidx1, didx1, rows1, semg1, sems1)

            @pl.when(i % 2 == 1)
            def _():
                step(i, sidx1, didx1, rows1, semg1, sems1,
                     sidx0, didx0, rows0, semg0, sems0)

            return 0

        lax.fori_loop(0, NITER, body, 0)
        pltpu.make_async_copy(rows1, acc.at[pl.ds(0, CH)], sems1).wait()
        pltpu.make_async_copy(rows0, acc.at[pl.ds(0, CH)], sems0).wait()
        plsc.subcore_barrier()

        for kk in range(RT // ZR):
            pltpu.sync_copy(acc.at[pl.ds(row0 + kk * ZR, ZR)], zbuf)
            pltpu.sync_copy(zbuf, out_hbm.at[cid, pl.ds(row0 + kk * ZR, ZR)])

    return k(h, src, dst)


def _tc_norms(deg_partials):
    def k(dp_ref, o_ref):
        deg = dp_ref[0] + dp_ref[1]
        o_ref[...] = lax.rsqrt(jnp.maximum(deg, 1.0))

    return pl.pallas_call(
        k,
        out_shape=jax.ShapeDtypeStruct((2, NPAD), jnp.float32),
    )(deg_partials)


def _tc_pre_matmul(x, ns, w):
    R = 1000

    def k(x_ref, ns_ref, w_ref, o_ref):
        o_ref[...] = jnp.dot(x_ref[...] * ns_ref[...], w_ref[...],
                             preferred_element_type=jnp.float32)

    d_in, d_out = w.shape
    return pl.pallas_call(
        k,
        grid=(N // R,),
        in_specs=[
            pl.BlockSpec((R, d_in), lambda i: (i, 0)),
            pl.BlockSpec((R, 1), lambda i: (i, 0)),
            pl.BlockSpec((d_in, d_out), lambda i: (0, 0)),
        ],
        out_specs=pl.BlockSpec((R, d_out), lambda i: (i, 0)),
        out_shape=jax.ShapeDtypeStruct((N, d_out), jnp.float32),
    )(x, ns, w)


def _tc_mid(partials, nd, ns, b, w):
    R = 1000

    def k(p_ref, nd_ref, ns_ref, b_ref, w_ref, o_ref):
        t = (p_ref[0] + p_ref[1]) * nd_ref[...] + b_ref[...]
        t = jnp.maximum(t, 0.0) * ns_ref[...]
        o_ref[...] = jnp.dot(t, w_ref[...], preferred_element_type=jnp.float32)

    d_in, d_out = w.shape
    return pl.pallas_call(
        k,
        grid=(N // R,),
        in_specs=[
            pl.BlockSpec((NC, R, d_in), lambda i: (0, i, 0)),
            pl.BlockSpec((R, 1), lambda i: (i, 0)),
            pl.BlockSpec((R, 1), lambda i: (i, 0)),
            pl.BlockSpec((1, d_in), lambda i: (0, 0)),
            pl.BlockSpec((d_in, d_out), lambda i: (0, 0)),
        ],
        out_specs=pl.BlockSpec((R, d_out), lambda i: (i, 0)),
        out_shape=jax.ShapeDtypeStruct((N, d_out), jnp.float32),
    )(partials, nd, ns, b, w)


def _tc_elem(partials, nd, ns, b):
    R = 1000

    def k(p_ref, nd_ref, ns_ref, b_ref, o_ref):
        t = (p_ref[0] + p_ref[1]) * nd_ref[...] + b_ref[...]
        o_ref[...] = jnp.maximum(t, 0.0) * ns_ref[...]

    return pl.pallas_call(
        k,
        grid=(N // R,),
        in_specs=[
            pl.BlockSpec((NC, R, D_H), lambda i: (0, i, 0)),
            pl.BlockSpec((R, 1), lambda i: (i, 0)),
            pl.BlockSpec((R, 1), lambda i: (i, 0)),
            pl.BlockSpec((1, D_H), lambda i: (0, 0)),
        ],
        out_specs=pl.BlockSpec((R, D_H), lambda i: (i, 0)),
        out_shape=jax.ShapeDtypeStruct((N, D_H), jnp.float32),
    )(partials, nd, ns, b)


def _tc_final_matmul(partials, nd, b, w):
    R = 1000

    def k(p_ref, nd_ref, b_ref, w_ref, o_ref):
        t = (p_ref[0] + p_ref[1]) * nd_ref[...]
        o_ref[...] = jnp.dot(t, w_ref[...],
                             preferred_element_type=jnp.float32) + b_ref[...]

    d_in, d_out = w.shape
    return pl.pallas_call(
        k,
        grid=(N // R,),
        in_specs=[
            pl.BlockSpec((NC, R, d_in), lambda i: (0, i, 0)),
            pl.BlockSpec((R, 1), lambda i: (i, 0)),
            pl.BlockSpec((1, d_out), lambda i: (0, 0)),
            pl.BlockSpec((d_in, d_out), lambda i: (0, 0)),
        ],
        out_specs=pl.BlockSpec((R, d_out), lambda i: (i, 0)),
        out_shape=jax.ShapeDtypeStruct((N, d_out), jnp.float32),
    )(partials, nd, b, w)


def kernel(x, edge_index, W1, b1, W2, b2, W3, b3):
    src = edge_index[0]
    dst = edge_index[1]

    deg_partials = _sc_degrees(src, dst)
    norms = _tc_norms(deg_partials)
    ns = norms[0, :N, None]
    nd = norms[1, :N, None]

    h = _tc_pre_matmul(x, ns, W1)
    p = _sc_spmm(h, src, dst, D_H)[:, :N]
    h = _tc_mid(p, nd, ns, b1[None, :], W2)
    p = _sc_spmm(h, src, dst, D_H)[:, :N]
    h = _tc_elem(p, nd, ns, b2[None, :])
    p = _sc_spmm(h, src, dst, D_H)[:, :N]
    return _tc_final_matmul(p, nd, b3[None, :], W3)
